# baseline (speedup 1.0000x reference)
import functools

import jax
import jax.numpy as jnp
from jax import lax
from jax.experimental import pallas as pl
from jax.experimental.pallas import tpu as pltpu
from jax.experimental.pallas import tpu_sc as plsc

NUM_EMBEDDINGS = 1000000
D = 64
B_TOTAL = 4096 * 200

NC, NS = 2, 16
NW = NC * NS
N_PER_W = B_TOTAL // NW
C = 128
NCH = N_PER_W // C
S = 8
A = 4

assert NCH % S == 0 and A < S


def _gather_start(table_hbm, idx_v, rows, gsems, g, slot):
    pltpu.make_async_copy(
        table_hbm.at[idx_v.at[pl.ds(g * C, C)]], rows[slot], gsems[slot]
    ).start()


def _gather_wait(table_hbm, idx_v, rows, gsems, g, slot):
    pltpu.make_async_copy(
        table_hbm.at[idx_v.at[pl.ds(g * C, C)]], rows[slot], gsems[slot]
    ).wait()


def _write_start(out_hbm, rows, wsems, base, g, slot):
    pltpu.make_async_copy(
        rows[slot], out_hbm.at[pl.ds(base + g * C, C)], wsems[slot]
    ).start()


def _write_wait(out_hbm, rows, wsems, base, g, slot):
    pltpu.make_async_copy(
        rows[slot], out_hbm.at[pl.ds(base + g * C, C)], wsems[slot]
    ).wait()


@functools.partial(
    pl.kernel,
    out_type=jax.ShapeDtypeStruct((B_TOTAL, D), jnp.float32),
    mesh=plsc.VectorSubcoreMesh(core_axis_name="c", subcore_axis_name="s"),
    compiler_params=pltpu.CompilerParams(use_tc_tiling_on_sc=False),
    scratch_types=[
        pltpu.VMEM((N_PER_W,), jnp.int32),
        [pltpu.VMEM((C, D), jnp.float32) for _ in range(S)],
        [pltpu.SemaphoreType.DMA for _ in range(S)],
        [pltpu.SemaphoreType.DMA for _ in range(S)],
    ],
)
def _embed_sc(idx_hbm, table_hbm, out_hbm, idx_v, rows, gsems, wsems):
    wid = lax.axis_index("s") * NC + lax.axis_index("c")
    base = wid * N_PER_W

    pltpu.sync_copy(idx_hbm.at[pl.ds(base, N_PER_W)], idx_v)

    for g in range(A):
        _gather_start(table_hbm, idx_v, rows, gsems, g, g)

    for g in range(S):
        if g + A < NCH:
            bb = (g + A) % S
            if g >= A:
                _write_wait(out_hbm, rows, wsems, base, g - A, bb)
            _gather_start(table_hbm, idx_v, rows, gsems, g + A, bb)
        _gather_wait(table_hbm, idx_v, rows, gsems, g, g % S)
        _write_start(out_hbm, rows, wsems, base, g, g % S)

    def trip(i, _):
        g0 = i * S
        for b in range(S):
            g = g0 + b
            bb = (b + A) % S
            _write_wait(out_hbm, rows, wsems, base, g - A, bb)
            _gather_start(table_hbm, idx_v, rows, gsems, g + A, bb)
            _gather_wait(table_hbm, idx_v, rows, gsems, g, b)
            _write_start(out_hbm, rows, wsems, base, g, b)
        return _

    lax.fori_loop(1, NCH // S - 1, trip, 0)

    g0 = NCH - S
    for b in range(S):
        g = g0 + b
        if g + A < NCH:
            bb = (b + A) % S
            _write_wait(out_hbm, rows, wsems, base, g - A, bb)
            _gather_start(table_hbm, idx_v, rows, gsems, g + A, bb)
        _gather_wait(table_hbm, idx_v, rows, gsems, g, b)
        _write_start(out_hbm, rows, wsems, base, g, b)

    for b in range(S):
        _write_wait(out_hbm, rows, wsems, base, g0 + b, b)


def kernel(token_ids, embedding_matrix):
    idx = token_ids.reshape(-1)
    out = _embed_sc(idx, embedding_matrix)
    return out.reshape(token_ids.shape[0], token_ids.shape[1], D)
